# baseline (speedup 1.0000x reference)
import dataclasses
import functools

import jax
import jax.numpy as jnp
from jax import lax
from jax.experimental import pallas as pl
from jax.experimental.pallas import tpu as pltpu
from jax.experimental.pallas import tpu_sc as plsc

H = 128
NC, NS, L = 2, 16, 16
NW = NC * NS
W = 128


def _sc_lookup_sum(word_table, ltab_hbm, idx_all, n, lt_words):
    rows_per_w = n // NW
    chunks = rows_per_w // W
    mesh = plsc.VectorSubcoreMesh(core_axis_name="c", subcore_axis_name="s")
    cp = pltpu.CompilerParams()
    if "needs_layout_passes" in pltpu.CompilerParams.__dataclass_fields__:
        cp = dataclasses.replace(cp, needs_layout_passes=False)

    @functools.partial(
        pl.kernel,
        out_type=jax.ShapeDtypeStruct((n, H), jnp.float32),
        mesh=mesh,
        compiler_params=cp,
        scratch_types=[
            pltpu.VMEM((6, W), jnp.int32),
            pltpu.VMEM((lt_words,), jnp.float32),
            pltpu.VMEM((W, H), jnp.float32),
            pltpu.VMEM((W, H), jnp.float32),
            pltpu.SemaphoreType.DMA,
            pltpu.SemaphoreType.DMA,
            pltpu.SemaphoreType.DMA,
            pltpu.SemaphoreType.DMA,
        ],
    )
    def sc_kernel(wt_hbm, lt_hbm, idx_hbm, out_hbm,
                  ib, ltab, acc0, acc1, g0, g1, o0, o1):
        wid = lax.axis_index("s") * NC + lax.axis_index("c")
        cbase = wid * chunks
        rbase = wid * rows_per_w
        accs = (acc0, acc1)
        gsems, osems = (g0, g1), (o0, o1)

        pltpu.sync_copy(lt_hbm, ltab)

        iota = lax.iota(jnp.int32, L)
        cvec = [iota + L * j for j in range(H // L)]

        def fetch_idx(b, ci):
            pltpu.sync_copy(idx_hbm.at[cbase + ci], ib.at[pl.ds(3 * b, 3)])

        def fire(b):
            pltpu.async_copy(wt_hbm.at[ib.at[3 * b]], accs[b], gsems[b])

        def wait_gather(b):
            pltpu.make_async_copy(wt_hbm.at[ib.at[3 * b]], accs[b],
                                  gsems[b]).wait()

        def write(b, ci):
            pltpu.async_copy(accs[b], out_hbm.at[pl.ds(rbase + ci * W, W)],
                             osems[b])

        def wait_write(b):
            pltpu.make_async_copy(accs[b], out_hbm.at[pl.ds(rbase, W)],
                                  osems[b]).wait()

        def compute(b):
            acc = accs[b]

            @pl.loop(0, W, step=L)
            def _(r0):
                a_off = ib[3 * b + 1, pl.ds(r0, L)]
                p_off = ib[3 * b + 2, pl.ds(r0, L)]
                for r in range(L):
                    sa = jnp.broadcast_to(a_off[r], (L,))
                    sp = jnp.broadcast_to(p_off[r], (L,))
                    for j in range(H // L):
                        va = plsc.load_gather(ltab, [sa + cvec[j]])
                        vp = plsc.load_gather(ltab, [sp + cvec[j]])
                        plsc.addupdate(acc.at[r0 + r, pl.ds(j * L, L)],
                                       va + vp)

        for b in (0, 1):
            fetch_idx(b, b)
            fire(b)

        @pl.loop(0, chunks - 2, step=2)
        def _(ci):
            for b in (0, 1):
                wait_gather(b)
                compute(b)
                write(b, ci + b)
            for b in (0, 1):
                wait_write(b)
                fetch_idx(b, ci + 2 + b)
                fire(b)

        for b in (0, 1):
            wait_gather(b)
            compute(b)
            write(b, chunks - 2 + b)
        for b in (0, 1):
            wait_write(b)

    return sc_kernel(word_table, ltab_hbm, idx_all)


def kernel(word_x, age_x, pos_x, word_table, age_table, pos_table):
    b, s = word_x.shape
    n = b * s
    av = age_table.shape[0]
    ltab = jnp.concatenate(
        [age_table.reshape(-1), pos_table.reshape(-1)]).astype(jnp.float32)
    a_off = age_x.astype(jnp.int32) * H
    p_off = (av + pos_x.astype(jnp.int32)) * H
    idx_all = jnp.stack(
        [word_x.reshape(-1, W).astype(jnp.int32),
         a_off.reshape(-1, W),
         p_off.reshape(-1, W)],
        axis=1)
    out = _sc_lookup_sum(word_table, ltab, idx_all, n, ltab.shape[0])
    return out.reshape(b, s, H)


# speedup vs baseline: 1.1453x; 1.1453x over previous
import dataclasses
import functools

import jax
import jax.numpy as jnp
from jax import lax
from jax.experimental import pallas as pl
from jax.experimental.pallas import tpu as pltpu
from jax.experimental.pallas import tpu_sc as plsc

H = 128
NC, NS, L = 2, 16, 16
NW = NC * NS
W = 128


def _sc_lookup_sum(word_table, ltab_hbm, idx_all, n, lt_words):
    rows_per_w = n // NW
    chunks = rows_per_w // W
    mesh = plsc.VectorSubcoreMesh(core_axis_name="c", subcore_axis_name="s")
    cp = pltpu.CompilerParams()
    if "needs_layout_passes" in pltpu.CompilerParams.__dataclass_fields__:
        cp = dataclasses.replace(cp, needs_layout_passes=False)

    @functools.partial(
        pl.kernel,
        out_type=jax.ShapeDtypeStruct((n, H), jnp.float32),
        mesh=mesh,
        compiler_params=cp,
        scratch_types=[
            pltpu.VMEM((6, W), jnp.int32),
            pltpu.VMEM((lt_words,), jnp.float32),
            pltpu.VMEM((W, H), jnp.float32),
            pltpu.VMEM((W, H), jnp.float32),
            pltpu.SemaphoreType.DMA,
            pltpu.SemaphoreType.DMA,
            pltpu.SemaphoreType.DMA,
            pltpu.SemaphoreType.DMA,
        ],
    )
    def sc_kernel(wt_hbm, lt_hbm, idx_hbm, out_hbm,
                  ib, ltab, acc0, acc1, g0, g1, o0, o1):
        wid = lax.axis_index("s") * NC + lax.axis_index("c")
        cbase = wid * chunks
        rbase = wid * rows_per_w
        accs = (acc0, acc1)
        gsems, osems = (g0, g1), (o0, o1)

        pltpu.sync_copy(lt_hbm, ltab)

        iota = lax.iota(jnp.int32, L)
        cvec = [iota + L * j for j in range(H // L)]

        def fetch_idx(b, ci):
            pltpu.sync_copy(idx_hbm.at[cbase + ci], ib.at[pl.ds(3 * b, 3)])

        def fire(b):
            pltpu.async_copy(wt_hbm.at[ib.at[3 * b]], accs[b], gsems[b])

        def wait_gather(b):
            pltpu.make_async_copy(wt_hbm.at[ib.at[3 * b]], accs[b],
                                  gsems[b]).wait()

        def write(b, ci):
            pltpu.async_copy(accs[b], out_hbm.at[pl.ds(rbase + ci * W, W)],
                             osems[b])

        def wait_write(b):
            pltpu.make_async_copy(accs[b], out_hbm.at[pl.ds(rbase, W)],
                                  osems[b]).wait()

        def compute(b):
            acc = accs[b]

            @plsc.parallel_loop(0, W, L)
            def _(r0):
                a_off = ib[3 * b + 1, pl.ds(r0, L)]
                p_off = ib[3 * b + 2, pl.ds(r0, L)]
                for r in range(L):
                    sa = jnp.broadcast_to(a_off[r], (L,))
                    sp = jnp.broadcast_to(p_off[r], (L,))
                    for j in range(H // L):
                        va = plsc.load_gather(ltab, [sa + cvec[j]])
                        vp = plsc.load_gather(ltab, [sp + cvec[j]])
                        plsc.addupdate(acc.at[r0 + r, pl.ds(j * L, L)],
                                       va + vp)

        for b in (0, 1):
            fetch_idx(b, b)
            fire(b)

        @pl.loop(0, chunks, step=2)
        def _(ci):
            for b in (0, 1):
                wait_gather(b)
                compute(b)
                write(b, ci + b)

            @pl.when(ci + 2 < chunks)
            def _():
                for b in (0, 1):
                    wait_write(b)
                    fetch_idx(b, ci + 2 + b)
                    fire(b)

        for b in (0, 1):
            wait_write(b)

    return sc_kernel(word_table, ltab_hbm, idx_all)


def kernel(word_x, age_x, pos_x, word_table, age_table, pos_table):
    b, s = word_x.shape
    n = b * s
    av = age_table.shape[0]
    ltab = jnp.concatenate(
        [age_table.reshape(-1), pos_table.reshape(-1)]).astype(jnp.float32)
    a_off = age_x.astype(jnp.int32) * H
    p_off = (av + pos_x.astype(jnp.int32)) * H
    idx_all = jnp.stack(
        [word_x.reshape(-1, W).astype(jnp.int32),
         a_off.reshape(-1, W),
         p_off.reshape(-1, W)],
        axis=1)
    out = _sc_lookup_sum(word_table, ltab, idx_all, n, ltab.shape[0])
    return out.reshape(b, s, H)


# speedup vs baseline: 1.7711x; 1.5464x over previous
import dataclasses
import functools

import jax
import jax.numpy as jnp
from jax import lax
from jax.experimental import pallas as pl
from jax.experimental.pallas import tpu as pltpu
from jax.experimental.pallas import tpu_sc as plsc

H = 128
NC, NS, L = 2, 16, 16
NW = NC * NS
W = 128


def _sc_lookup_sum(word_table, ltab_hbm, idx_all, n, lt_words):
    rows_per_w = n // NW
    chunks = rows_per_w // W
    mesh = plsc.VectorSubcoreMesh(core_axis_name="c", subcore_axis_name="s")
    cp = pltpu.CompilerParams()
    if "needs_layout_passes" in pltpu.CompilerParams.__dataclass_fields__:
        cp = dataclasses.replace(cp, needs_layout_passes=False)

    @functools.partial(
        pl.kernel,
        out_type=jax.ShapeDtypeStruct((n, H), jnp.float32),
        mesh=mesh,
        compiler_params=cp,
        scratch_types=[
            pltpu.VMEM((6, W), jnp.int32),
            pltpu.VMEM((lt_words,), jnp.float32),
            pltpu.VMEM((W, H), jnp.float32),
            pltpu.VMEM((W, H), jnp.float32),
            pltpu.SemaphoreType.DMA,
            pltpu.SemaphoreType.DMA,
            pltpu.SemaphoreType.DMA,
            pltpu.SemaphoreType.DMA,
        ],
    )
    def sc_kernel(wt_hbm, lt_hbm, idx_hbm, out_hbm,
                  ib, ltab, acc0, acc1, g0, g1, o0, o1):
        wid = lax.axis_index("s") * NC + lax.axis_index("c")
        cbase = wid * chunks
        rbase = wid * rows_per_w
        accs = (acc0, acc1)
        gsems, osems = (g0, g1), (o0, o1)

        pltpu.sync_copy(lt_hbm, ltab)

        iota = lax.iota(jnp.int32, L)
        cvec = [iota + L * j for j in range(H // L)]

        def fetch_idx(b, ci):
            pltpu.sync_copy(idx_hbm.at[cbase + ci], ib.at[pl.ds(3 * b, 3)])

        def fire(b):
            pltpu.async_copy(wt_hbm.at[ib.at[3 * b]], accs[b], gsems[b])

        def wait_gather(b):
            pltpu.make_async_copy(wt_hbm.at[ib.at[3 * b]], accs[b],
                                  gsems[b]).wait()

        def write(b, ci):
            pltpu.async_copy(accs[b], out_hbm.at[pl.ds(rbase + ci * W, W)],
                             osems[b])

        def wait_write(b):
            pltpu.make_async_copy(accs[b], out_hbm.at[pl.ds(rbase, W)],
                                  osems[b]).wait()

        def compute(b):
            acc = accs[b]

            IL = 4

            @plsc.parallel_loop(0, W, L)
            def _(r0):
                a_off = ib[3 * b + 1, pl.ds(r0, L)]
                p_off = ib[3 * b + 2, pl.ds(r0, L)]
                for r in range(0, L, IL):
                    sa = [jnp.broadcast_to(a_off[r + k], (L,))
                          for k in range(IL)]
                    sp = [jnp.broadcast_to(p_off[r + k], (L,))
                          for k in range(IL)]
                    for j in range(H // L):
                        vs = [plsc.load_gather(ltab, [sa[k] + cvec[j]]) +
                              plsc.load_gather(ltab, [sp[k] + cvec[j]])
                              for k in range(IL)]
                        for k in range(IL):
                            plsc.addupdate(
                                acc.at[r0 + r + k, pl.ds(j * L, L)], vs[k])

        for b in (0, 1):
            fetch_idx(b, b)
            fire(b)

        @pl.loop(0, chunks, step=2)
        def _(ci):
            for b in (0, 1):
                wait_gather(b)
                compute(b)
                write(b, ci + b)

            @pl.when(ci + 2 < chunks)
            def _():
                for b in (0, 1):
                    wait_write(b)
                    fetch_idx(b, ci + 2 + b)
                    fire(b)

        for b in (0, 1):
            wait_write(b)

    return sc_kernel(word_table, ltab_hbm, idx_all)


def kernel(word_x, age_x, pos_x, word_table, age_table, pos_table):
    b, s = word_x.shape
    n = b * s
    av = age_table.shape[0]
    ltab = jnp.concatenate(
        [age_table.reshape(-1), pos_table.reshape(-1)]).astype(jnp.float32)
    a_off = age_x.astype(jnp.int32) * H
    p_off = (av + pos_x.astype(jnp.int32)) * H
    idx_all = jnp.stack(
        [word_x.reshape(-1, W).astype(jnp.int32),
         a_off.reshape(-1, W),
         p_off.reshape(-1, W)],
        axis=1)
    out = _sc_lookup_sum(word_table, ltab, idx_all, n, ltab.shape[0])
    return out.reshape(b, s, H)


# speedup vs baseline: 1.8487x; 1.0438x over previous
import dataclasses
import functools

import jax
import jax.numpy as jnp
from jax import lax
from jax.experimental import pallas as pl
from jax.experimental.pallas import tpu as pltpu
from jax.experimental.pallas import tpu_sc as plsc

H = 128
NC, NS, L = 2, 16, 16
NW = NC * NS
W = 128


def _sc_lookup_sum(word_table, ltab_hbm, idx_all, n, lt_words):
    rows_per_w = n // NW
    chunks = rows_per_w // W
    mesh = plsc.VectorSubcoreMesh(core_axis_name="c", subcore_axis_name="s")
    cp = pltpu.CompilerParams()
    if "needs_layout_passes" in pltpu.CompilerParams.__dataclass_fields__:
        cp = dataclasses.replace(cp, needs_layout_passes=False)

    @functools.partial(
        pl.kernel,
        out_type=jax.ShapeDtypeStruct((n, H), jnp.float32),
        mesh=mesh,
        compiler_params=cp,
        scratch_types=[
            pltpu.VMEM((6, W), jnp.int32),
            pltpu.VMEM((lt_words,), jnp.float32),
            pltpu.VMEM((W, H), jnp.float32),
            pltpu.VMEM((W, H), jnp.float32),
            pltpu.SemaphoreType.DMA,
            pltpu.SemaphoreType.DMA,
            pltpu.SemaphoreType.DMA,
            pltpu.SemaphoreType.DMA,
        ],
    )
    def sc_kernel(wt_hbm, lt_hbm, idx_hbm, out_hbm,
                  ib, ltab, acc0, acc1, g0, g1, o0, o1):
        wid = lax.axis_index("s") * NC + lax.axis_index("c")
        cbase = wid * chunks
        rbase = wid * rows_per_w
        accs = (acc0, acc1)
        gsems, osems = (g0, g1), (o0, o1)

        pltpu.sync_copy(lt_hbm, ltab)

        iota = lax.iota(jnp.int32, L)
        cvec = [iota + L * j for j in range(H // L)]

        def fetch_idx(b, ci):
            pltpu.sync_copy(idx_hbm.at[cbase + ci], ib.at[pl.ds(3 * b, 3)])

        def fire(b):
            pltpu.async_copy(wt_hbm.at[ib.at[3 * b]], accs[b], gsems[b])

        def wait_gather(b):
            pltpu.make_async_copy(wt_hbm.at[ib.at[3 * b]], accs[b],
                                  gsems[b]).wait()

        def write(b, ci):
            pltpu.async_copy(accs[b], out_hbm.at[pl.ds(rbase + ci * W, W)],
                             osems[b])

        def wait_write(b):
            pltpu.make_async_copy(accs[b], out_hbm.at[pl.ds(rbase, W)],
                                  osems[b]).wait()

        def compute(b):
            acc = accs[b]

            IL = 8

            @plsc.parallel_loop(0, W, L)
            def _(r0):
                a_off = ib[3 * b + 1, pl.ds(r0, L)]
                p_off = ib[3 * b + 2, pl.ds(r0, L)]
                for r in range(0, L, IL):
                    sa = [jnp.broadcast_to(a_off[r + k], (L,))
                          for k in range(IL)]
                    sp = [jnp.broadcast_to(p_off[r + k], (L,))
                          for k in range(IL)]
                    for j in range(H // L):
                        vs = [plsc.load_gather(ltab, [sa[k] + cvec[j]]) +
                              plsc.load_gather(ltab, [sp[k] + cvec[j]])
                              for k in range(IL)]
                        for k in range(IL):
                            plsc.addupdate(
                                acc.at[r0 + r + k, pl.ds(j * L, L)], vs[k])

        for b in (0, 1):
            fetch_idx(b, b)
            fire(b)

        @pl.loop(0, chunks, step=2)
        def _(ci):
            for b in (0, 1):
                wait_gather(b)
                compute(b)
                write(b, ci + b)

            @pl.when(ci + 2 < chunks)
            def _():
                for b in (0, 1):
                    wait_write(b)
                    fetch_idx(b, ci + 2 + b)
                    fire(b)

        for b in (0, 1):
            wait_write(b)

    return sc_kernel(word_table, ltab_hbm, idx_all)


def kernel(word_x, age_x, pos_x, word_table, age_table, pos_table):
    b, s = word_x.shape
    n = b * s
    av = age_table.shape[0]
    ltab = jnp.concatenate(
        [age_table.reshape(-1), pos_table.reshape(-1)]).astype(jnp.float32)
    a_off = age_x.astype(jnp.int32) * H
    p_off = (av + pos_x.astype(jnp.int32)) * H
    idx_all = jnp.stack(
        [word_x.reshape(-1, W).astype(jnp.int32),
         a_off.reshape(-1, W),
         p_off.reshape(-1, W)],
        axis=1)
    out = _sc_lookup_sum(word_table, ltab, idx_all, n, ltab.shape[0])
    return out.reshape(b, s, H)


# speedup vs baseline: 2.3665x; 1.2801x over previous
import dataclasses
import functools

import jax
import jax.numpy as jnp
from jax import lax
from jax.experimental import pallas as pl
from jax.experimental.pallas import tpu as pltpu
from jax.experimental.pallas import tpu_sc as plsc

H = 128
NC, NS, L = 2, 16, 16
NW = NC * NS
W = 128


def _sc_lookup_sum(word_table, ltab_hbm, idx_all, n, lt_words):
    rows_per_w = n // NW
    chunks = rows_per_w // W
    mesh = plsc.VectorSubcoreMesh(core_axis_name="c", subcore_axis_name="s")
    cp = pltpu.CompilerParams()
    if "needs_layout_passes" in pltpu.CompilerParams.__dataclass_fields__:
        cp = dataclasses.replace(cp, needs_layout_passes=False)

    @functools.partial(
        pl.kernel,
        out_type=jax.ShapeDtypeStruct((n, H), jnp.float32),
        mesh=mesh,
        compiler_params=cp,
        scratch_types=[
            pltpu.VMEM((6, W), jnp.int32),
            pltpu.VMEM((lt_words,), jnp.int32),
            pltpu.VMEM((W, H), jnp.float32),
            pltpu.VMEM((W, H), jnp.float32),
            pltpu.SemaphoreType.DMA,
            pltpu.SemaphoreType.DMA,
            pltpu.SemaphoreType.DMA,
            pltpu.SemaphoreType.DMA,
        ],
    )
    def sc_kernel(wt_hbm, lt_hbm, idx_hbm, out_hbm,
                  ib, ltab, acc0, acc1, g0, g1, o0, o1):
        wid = lax.axis_index("s") * NC + lax.axis_index("c")
        cbase = wid * chunks
        rbase = wid * rows_per_w
        accs = (acc0, acc1)
        gsems, osems = (g0, g1), (o0, o1)

        pltpu.sync_copy(lt_hbm, ltab)

        iota = lax.iota(jnp.int32, L)
        lslices = [ltab.at[pl.ds(L * j, lt_words - L * j)]
                   for j in range(H // (2 * L))]

        def fetch_idx(b, ci):
            pltpu.sync_copy(idx_hbm.at[cbase + ci], ib.at[pl.ds(3 * b, 3)])

        def fire(b):
            pltpu.async_copy(wt_hbm.at[ib.at[3 * b]], accs[b], gsems[b])

        def wait_gather(b):
            pltpu.make_async_copy(wt_hbm.at[ib.at[3 * b]], accs[b],
                                  gsems[b]).wait()

        def write(b, ci):
            pltpu.async_copy(accs[b], out_hbm.at[pl.ds(rbase + ci * W, W)],
                             osems[b])

        def wait_write(b):
            pltpu.make_async_copy(accs[b], out_hbm.at[pl.ds(rbase, W)],
                                  osems[b]).wait()

        def compute(b):
            acc = accs[b]

            IL = 8

            @plsc.parallel_loop(0, W, L)
            def _(r0):
                a_off = ib[3 * b + 1, pl.ds(r0, L)]
                p_off = ib[3 * b + 2, pl.ds(r0, L)]
                for r in range(0, L, IL):
                    sa = [jnp.broadcast_to(a_off[r + k], (L,)) + iota
                          for k in range(IL)]
                    sp = [jnp.broadcast_to(p_off[r + k], (L,)) + iota
                          for k in range(IL)]
                    for j in range(H // (2 * L)):
                        vs = []
                        for k in range(IL):
                            ga = plsc.load_gather(lslices[j], [sa[k]])
                            gp = plsc.load_gather(lslices[j], [sp[k]])
                            vs.append(plsc.bitcast(ga, jnp.bfloat16) +
                                      plsc.bitcast(gp, jnp.bfloat16))
                        for k in range(IL):
                            lo16, hi16 = plsc.unpack(
                                vs[k], format=plsc.PackFormat.INTERLEAVED)
                            plsc.addupdate(
                                acc.at[r0 + r + k, pl.ds(2 * j * L, L)],
                                lo16)
                            plsc.addupdate(
                                acc.at[r0 + r + k, pl.ds(2 * j * L + L, L)],
                                hi16)

        for b in (0, 1):
            fetch_idx(b, b)
            fire(b)

        @pl.loop(0, chunks, step=2)
        def _(ci):
            for b in (0, 1):
                wait_gather(b)
                compute(b)
                write(b, ci + b)

            @pl.when(ci + 2 < chunks)
            def _():
                for b in (0, 1):
                    wait_write(b)
                    fetch_idx(b, ci + 2 + b)
                    fire(b)

        for b in (0, 1):
            wait_write(b)

    return sc_kernel(word_table, ltab_hbm, idx_all)


def kernel(word_x, age_x, pos_x, word_table, age_table, pos_table):
    b, s = word_x.shape
    n = b * s
    av = age_table.shape[0]
    wpr = H // 2
    rows = jnp.concatenate([age_table, pos_table]).astype(jnp.bfloat16)
    r4 = rows.reshape(-1, H // (2 * L), 2, L)
    lo = lax.bitcast_convert_type(r4[:, :, 0, :], jnp.uint16)
    hi = lax.bitcast_convert_type(r4[:, :, 1, :], jnp.uint16)
    words = lo.astype(jnp.uint32) | (hi.astype(jnp.uint32) << 16)
    ltab = lax.bitcast_convert_type(words, jnp.int32).reshape(-1)
    a_off = age_x.astype(jnp.int32) * wpr
    p_off = (av + pos_x.astype(jnp.int32)) * wpr
    idx_all = jnp.stack(
        [word_x.reshape(-1, W).astype(jnp.int32),
         a_off.reshape(-1, W),
         p_off.reshape(-1, W)],
        axis=1)
    out = _sc_lookup_sum(word_table, ltab, idx_all, n, ltab.shape[0])
    return out.reshape(b, s, H)


# speedup vs baseline: 2.5455x; 1.0756x over previous
import dataclasses
import functools

import jax
import jax.numpy as jnp
from jax import lax
from jax.experimental import pallas as pl
from jax.experimental.pallas import tpu as pltpu
from jax.experimental.pallas import tpu_sc as plsc

H = 128
NC, NS, L = 2, 16, 16
NW = NC * NS
W = 128


def _sc_lookup_sum(word_table, ltab_hbm, idx_all, n, lt_words):
    rows_per_w = n // NW
    chunks = rows_per_w // W
    mesh = plsc.VectorSubcoreMesh(core_axis_name="c", subcore_axis_name="s")
    cp = pltpu.CompilerParams()
    if "needs_layout_passes" in pltpu.CompilerParams.__dataclass_fields__:
        cp = dataclasses.replace(cp, needs_layout_passes=False)

    @functools.partial(
        pl.kernel,
        out_type=jax.ShapeDtypeStruct((n, H), jnp.float32),
        mesh=mesh,
        compiler_params=cp,
        scratch_types=[
            pltpu.VMEM((6, W), jnp.int32),
            pltpu.VMEM((lt_words,), jnp.int32),
            pltpu.VMEM((W, H), jnp.float32),
            pltpu.VMEM((W, H), jnp.float32),
            pltpu.SemaphoreType.DMA,
            pltpu.SemaphoreType.DMA,
            pltpu.SemaphoreType.DMA,
            pltpu.SemaphoreType.DMA,
            pltpu.SemaphoreType.DMA,
            pltpu.SemaphoreType.DMA,
        ],
    )
    def sc_kernel(wt_hbm, lt_hbm, idx_hbm, out_hbm,
                  ib, ltab, acc0, acc1, g0, g1, o0, o1, i0, i1):
        wid = lax.axis_index("s") * NC + lax.axis_index("c")
        cbase = wid * chunks
        rbase = wid * rows_per_w
        accs = (acc0, acc1)
        gsems, osems, isems = (g0, g1), (o0, o1), (i0, i1)

        pltpu.sync_copy(lt_hbm, ltab)

        iota = lax.iota(jnp.int32, L)
        lslices = [ltab.at[pl.ds(L * j, lt_words - L * j)]
                   for j in range(H // (2 * L))]

        def fetch_idx(b, ci):
            pltpu.async_copy(idx_hbm.at[cbase + ci], ib.at[pl.ds(3 * b, 3)],
                             isems[b])

        def wait_idx(b):
            pltpu.make_async_copy(idx_hbm.at[cbase], ib.at[pl.ds(3 * b, 3)],
                                  isems[b]).wait()

        def fire(b):
            pltpu.async_copy(wt_hbm.at[ib.at[3 * b]], accs[b], gsems[b])

        def wait_gather(b):
            pltpu.make_async_copy(wt_hbm.at[ib.at[3 * b]], accs[b],
                                  gsems[b]).wait()

        def write(b, ci):
            pltpu.async_copy(accs[b], out_hbm.at[pl.ds(rbase + ci * W, W)],
                             osems[b])

        def wait_write(b):
            pltpu.make_async_copy(accs[b], out_hbm.at[pl.ds(rbase, W)],
                                  osems[b]).wait()

        def compute(b):
            acc = accs[b]

            IL = 8

            @plsc.parallel_loop(0, W, L)
            def _(r0):
                a_off = ib[3 * b + 1, pl.ds(r0, L)]
                p_off = ib[3 * b + 2, pl.ds(r0, L)]
                for r in range(0, L, IL):
                    sa = [jnp.broadcast_to(a_off[r + k], (L,)) + iota
                          for k in range(IL)]
                    sp = [jnp.broadcast_to(p_off[r + k], (L,)) + iota
                          for k in range(IL)]
                    for j in range(H // (2 * L)):
                        vs = []
                        for k in range(IL):
                            ga = plsc.load_gather(lslices[j], [sa[k]])
                            gp = plsc.load_gather(lslices[j], [sp[k]])
                            vs.append(plsc.bitcast(ga, jnp.bfloat16) +
                                      plsc.bitcast(gp, jnp.bfloat16))
                        for k in range(IL):
                            lo16, hi16 = plsc.unpack(
                                vs[k], format=plsc.PackFormat.INTERLEAVED)
                            plsc.addupdate(
                                acc.at[r0 + r + k, pl.ds(2 * j * L, L)],
                                lo16)
                            plsc.addupdate(
                                acc.at[r0 + r + k, pl.ds(2 * j * L + L, L)],
                                hi16)

        for b in (0, 1):
            fetch_idx(b, b)
            wait_idx(b)
            fire(b)

        @pl.loop(0, chunks, step=2)
        def _(ci):
            for b in (0, 1):
                wait_gather(b)
                compute(b)

                @pl.when(ci + 2 < chunks)
                def _():
                    fetch_idx(b, ci + 2 + b)

                write(b, ci + b)

            @pl.when(ci + 2 < chunks)
            def _():
                for b in (0, 1):
                    wait_write(b)
                    wait_idx(b)
                    fire(b)

        for b in (0, 1):
            wait_write(b)

    return sc_kernel(word_table, ltab_hbm, idx_all)


def kernel(word_x, age_x, pos_x, word_table, age_table, pos_table):
    b, s = word_x.shape
    n = b * s
    av = age_table.shape[0]
    wpr = H // 2
    rows = jnp.concatenate([age_table, pos_table]).astype(jnp.bfloat16)
    r4 = rows.reshape(-1, H // (2 * L), 2, L)
    lo = lax.bitcast_convert_type(r4[:, :, 0, :], jnp.uint16)
    hi = lax.bitcast_convert_type(r4[:, :, 1, :], jnp.uint16)
    words = lo.astype(jnp.uint32) | (hi.astype(jnp.uint32) << 16)
    ltab = lax.bitcast_convert_type(words, jnp.int32).reshape(-1)
    a_off = age_x.astype(jnp.int32) * wpr
    p_off = (av + pos_x.astype(jnp.int32)) * wpr
    idx_all = jnp.stack(
        [word_x.reshape(-1, W).astype(jnp.int32),
         a_off.reshape(-1, W),
         p_off.reshape(-1, W)],
        axis=1)
    out = _sc_lookup_sum(word_table, ltab, idx_all, n, ltab.shape[0])
    return out.reshape(b, s, H)


# speedup vs baseline: 2.8196x; 1.1077x over previous
import dataclasses
import functools

import jax
import jax.numpy as jnp
from jax import lax
from jax.experimental import pallas as pl
from jax.experimental.pallas import tpu as pltpu
from jax.experimental.pallas import tpu_sc as plsc

H = 128
NC, NS, L = 2, 16, 16
NW = NC * NS
W = 128


def _sc_lookup_sum(word_table, ltab_hbm, idx_all, n, lt_words):
    rows_per_w = n // NW
    chunks = rows_per_w // W
    mesh = plsc.VectorSubcoreMesh(core_axis_name="c", subcore_axis_name="s")
    cp = pltpu.CompilerParams()
    if "needs_layout_passes" in pltpu.CompilerParams.__dataclass_fields__:
        cp = dataclasses.replace(cp, needs_layout_passes=False)

    @functools.partial(
        pl.kernel,
        out_type=jax.ShapeDtypeStruct((n, H), jnp.float32),
        mesh=mesh,
        compiler_params=cp,
        scratch_types=[
            pltpu.VMEM((9, W), jnp.int32),
            pltpu.VMEM((lt_words,), jnp.int32),
            pltpu.VMEM((W, H), jnp.float32),
            pltpu.VMEM((W, H), jnp.float32),
            pltpu.VMEM((W, H), jnp.float32),
            pltpu.SemaphoreType.DMA,
            pltpu.SemaphoreType.DMA,
            pltpu.SemaphoreType.DMA,
            pltpu.SemaphoreType.DMA,
            pltpu.SemaphoreType.DMA,
            pltpu.SemaphoreType.DMA,
            pltpu.SemaphoreType.DMA,
            pltpu.SemaphoreType.DMA,
            pltpu.SemaphoreType.DMA,
        ],
    )
    def sc_kernel(wt_hbm, lt_hbm, idx_hbm, out_hbm,
                  ib, ltab, acc0, acc1, acc2,
                  g0, g1, g2, o0, o1, o2, i0, i1, i2):
        wid = lax.axis_index("s") * NC + lax.axis_index("c")
        cbase = wid * chunks
        rbase = wid * rows_per_w
        accs = (acc0, acc1, acc2)
        gsems, osems, isems = (g0, g1, g2), (o0, o1, o2), (i0, i1, i2)

        pltpu.sync_copy(lt_hbm, ltab)

        iota = lax.iota(jnp.int32, L)
        lslices = [ltab.at[pl.ds(L * j, lt_words - L * j)]
                   for j in range(H // (2 * L))]

        def fetch_idx(b, ci):
            pltpu.async_copy(idx_hbm.at[cbase + ci], ib.at[pl.ds(3 * b, 3)],
                             isems[b])

        def wait_idx(b):
            pltpu.make_async_copy(idx_hbm.at[cbase], ib.at[pl.ds(3 * b, 3)],
                                  isems[b]).wait()

        def fire(b):
            pltpu.async_copy(wt_hbm.at[ib.at[3 * b]], accs[b], gsems[b])

        def wait_gather(b):
            pltpu.make_async_copy(wt_hbm.at[ib.at[3 * b]], accs[b],
                                  gsems[b]).wait()

        def write(b, ci):
            pltpu.async_copy(accs[b], out_hbm.at[pl.ds(rbase + ci * W, W)],
                             osems[b])

        def wait_write(b):
            pltpu.make_async_copy(accs[b], out_hbm.at[pl.ds(rbase, W)],
                                  osems[b]).wait()

        def compute(b):
            acc = accs[b]

            IL = 8

            @plsc.parallel_loop(0, W, L)
            def _(r0):
                a_off = ib[3 * b + 1, pl.ds(r0, L)]
                p_off = ib[3 * b + 2, pl.ds(r0, L)]
                for r in range(0, L, IL):
                    sa = [jnp.broadcast_to(a_off[r + k], (L,)) + iota
                          for k in range(IL)]
                    sp = [jnp.broadcast_to(p_off[r + k], (L,)) + iota
                          for k in range(IL)]
                    for j in range(H // (2 * L)):
                        vs = []
                        for k in range(IL):
                            ga = plsc.load_gather(lslices[j], [sa[k]])
                            gp = plsc.load_gather(lslices[j], [sp[k]])
                            vs.append(plsc.bitcast(ga, jnp.bfloat16) +
                                      plsc.bitcast(gp, jnp.bfloat16))
                        for k in range(IL):
                            lo16, hi16 = plsc.unpack(
                                vs[k], format=plsc.PackFormat.INTERLEAVED)
                            plsc.addupdate(
                                acc.at[r0 + r + k, pl.ds(2 * j * L, L)],
                                lo16)
                            plsc.addupdate(
                                acc.at[r0 + r + k, pl.ds(2 * j * L + L, L)],
                                hi16)

        for b in (0, 1, 2):
            fetch_idx(b, b)
        for b in (0, 1):
            wait_idx(b)
            fire(b)

        main = chunks - (chunks % 3 or 3)

        @pl.loop(0, main, step=3)
        def _(ci):
            for k in (0, 1, 2):
                c = ci + k
                wait_gather(k)
                compute(k)

                @pl.when(c + 3 < chunks)
                def _():
                    fetch_idx(k, c + 3)

                write(k, c)
                bn = (k + 2) % 3

                @pl.when(c + 2 < chunks)
                def _():
                    @pl.when(c > 0)
                    def _():
                        wait_write(bn)

                    wait_idx(bn)
                    fire(bn)

        @pl.loop(main, chunks)
        def _(c):
            for k in (0, 1, 2):
                @pl.when(c % 3 == k)
                def _():
                    wait_gather(k)
                    compute(k)
                    write(k, c)
                    bn = (k + 2) % 3

                    @pl.when(c + 2 < chunks)
                    def _():
                        wait_write(bn)
                        wait_idx(bn)
                        fire(bn)

        for b in (0, 1, 2):
            wait_write(b)

    return sc_kernel(word_table, ltab_hbm, idx_all)


def kernel(word_x, age_x, pos_x, word_table, age_table, pos_table):
    b, s = word_x.shape
    n = b * s
    av = age_table.shape[0]
    wpr = H // 2
    rows = jnp.concatenate([age_table, pos_table]).astype(jnp.bfloat16)
    r4 = rows.reshape(-1, H // (2 * L), 2, L)
    lo = lax.bitcast_convert_type(r4[:, :, 0, :], jnp.uint16)
    hi = lax.bitcast_convert_type(r4[:, :, 1, :], jnp.uint16)
    words = lo.astype(jnp.uint32) | (hi.astype(jnp.uint32) << 16)
    ltab = lax.bitcast_convert_type(words, jnp.int32).reshape(-1)
    a_off = age_x.astype(jnp.int32) * wpr
    p_off = (av + pos_x.astype(jnp.int32)) * wpr
    idx_all = jnp.stack(
        [word_x.reshape(-1, W).astype(jnp.int32),
         a_off.reshape(-1, W),
         p_off.reshape(-1, W)],
        axis=1)
    out = _sc_lookup_sum(word_table, ltab, idx_all, n, ltab.shape[0])
    return out.reshape(b, s, H)


# speedup vs baseline: 2.8498x; 1.0107x over previous
import dataclasses
import functools

import jax
import jax.numpy as jnp
from jax import lax
from jax.experimental import pallas as pl
from jax.experimental.pallas import tpu as pltpu
from jax.experimental.pallas import tpu_sc as plsc

H = 128
NC, NS, L = 2, 16, 16
NW = NC * NS
W = 128


def _sc_lookup_sum(word_table, ltab_hbm, idx_all, n, lt_words):
    rows_per_w = n // NW
    chunks = rows_per_w // W
    mesh = plsc.VectorSubcoreMesh(core_axis_name="c", subcore_axis_name="s")
    cp = pltpu.CompilerParams()
    if "needs_layout_passes" in pltpu.CompilerParams.__dataclass_fields__:
        cp = dataclasses.replace(cp, needs_layout_passes=False)

    @functools.partial(
        pl.kernel,
        out_type=jax.ShapeDtypeStruct((n, H), jnp.float32),
        mesh=mesh,
        compiler_params=cp,
        scratch_types=[
            pltpu.VMEM((9, W), jnp.int32),
            pltpu.VMEM((lt_words,), jnp.int32),
            pltpu.VMEM((W, H), jnp.float32),
            pltpu.VMEM((W, H), jnp.float32),
            pltpu.VMEM((W, H), jnp.float32),
            pltpu.SemaphoreType.DMA,
            pltpu.SemaphoreType.DMA,
            pltpu.SemaphoreType.DMA,
            pltpu.SemaphoreType.DMA,
            pltpu.SemaphoreType.DMA,
            pltpu.SemaphoreType.DMA,
            pltpu.SemaphoreType.DMA,
            pltpu.SemaphoreType.DMA,
            pltpu.SemaphoreType.DMA,
        ],
    )
    def sc_kernel(wt_hbm, lt_hbm, idx_hbm, out_hbm,
                  ib, ltab, acc0, acc1, acc2,
                  g0, g1, g2, o0, o1, o2, i0, i1, i2):
        wid = lax.axis_index("s") * NC + lax.axis_index("c")
        cbase = wid * chunks
        rbase = wid * rows_per_w
        accs = (acc0, acc1, acc2)
        gsems, osems, isems = (g0, g1, g2), (o0, o1, o2), (i0, i1, i2)

        iota = lax.iota(jnp.int32, L)
        lslices = [ltab.at[pl.ds(L * j, lt_words - L * j)]
                   for j in range(H // (2 * L))]

        def fetch_idx(b, ci):
            pltpu.async_copy(idx_hbm.at[cbase + ci], ib.at[pl.ds(3 * b, 3)],
                             isems[b])

        def wait_idx(b):
            pltpu.make_async_copy(idx_hbm.at[cbase], ib.at[pl.ds(3 * b, 3)],
                                  isems[b]).wait()

        def fire(b):
            pltpu.async_copy(wt_hbm.at[ib.at[3 * b]], accs[b], gsems[b])

        def wait_gather(b):
            pltpu.make_async_copy(wt_hbm.at[ib.at[3 * b]], accs[b],
                                  gsems[b]).wait()

        def write(b, ci):
            pltpu.async_copy(accs[b], out_hbm.at[pl.ds(rbase + ci * W, W)],
                             osems[b])

        def wait_write(b):
            pltpu.make_async_copy(accs[b], out_hbm.at[pl.ds(rbase, W)],
                                  osems[b]).wait()

        def compute(b):
            acc = accs[b]

            IL = 8

            @plsc.parallel_loop(0, W, L)
            def _(r0):
                a_off = ib[3 * b + 1, pl.ds(r0, L)]
                p_off = ib[3 * b + 2, pl.ds(r0, L)]
                for r in range(0, L, IL):
                    sa = [jnp.broadcast_to(a_off[r + k], (L,)) + iota
                          for k in range(IL)]
                    sp = [jnp.broadcast_to(p_off[r + k], (L,)) + iota
                          for k in range(IL)]
                    for j in range(H // (2 * L)):
                        vs = []
                        for k in range(IL):
                            ga = plsc.load_gather(lslices[j], [sa[k]])
                            gp = plsc.load_gather(lslices[j], [sp[k]])
                            vs.append(plsc.bitcast(ga, jnp.bfloat16) +
                                      plsc.bitcast(gp, jnp.bfloat16))
                        for k in range(IL):
                            lo16, hi16 = plsc.unpack(
                                vs[k], format=plsc.PackFormat.INTERLEAVED)
                            plsc.addupdate(
                                acc.at[r0 + r + k, pl.ds(2 * j * L, L)],
                                lo16)
                            plsc.addupdate(
                                acc.at[r0 + r + k, pl.ds(2 * j * L + L, L)],
                                hi16)

        for b in (0, 1, 2):
            fetch_idx(b, b)
        for b in (0, 1):
            wait_idx(b)
            fire(b)
        pltpu.sync_copy(lt_hbm, ltab)

        main = chunks - (chunks % 3 or 3)

        @pl.loop(0, main, step=3)
        def _(ci):
            for k in (0, 1, 2):
                c = ci + k
                wait_gather(k)
                compute(k)

                @pl.when(c + 3 < chunks)
                def _():
                    fetch_idx(k, c + 3)

                write(k, c)
                bn = (k + 2) % 3

                @pl.when(c + 2 < chunks)
                def _():
                    @pl.when(c > 0)
                    def _():
                        wait_write(bn)

                    wait_idx(bn)
                    fire(bn)

        @pl.loop(main, chunks)
        def _(c):
            for k in (0, 1, 2):
                @pl.when(c % 3 == k)
                def _():
                    wait_gather(k)
                    compute(k)
                    write(k, c)
                    bn = (k + 2) % 3

                    @pl.when(c + 2 < chunks)
                    def _():
                        wait_write(bn)
                        wait_idx(bn)
                        fire(bn)

        for b in (0, 1, 2):
            wait_write(b)

    return sc_kernel(word_table, ltab_hbm, idx_all)


def kernel(word_x, age_x, pos_x, word_table, age_table, pos_table):
    b, s = word_x.shape
    n = b * s
    av = age_table.shape[0]
    wpr = H // 2
    rows = jnp.concatenate([age_table, pos_table]).astype(jnp.bfloat16)
    r4 = rows.reshape(-1, H // (2 * L), 2, L)
    lo = lax.bitcast_convert_type(r4[:, :, 0, :], jnp.uint16)
    hi = lax.bitcast_convert_type(r4[:, :, 1, :], jnp.uint16)
    words = lo.astype(jnp.uint32) | (hi.astype(jnp.uint32) << 16)
    ltab = lax.bitcast_convert_type(words, jnp.int32).reshape(-1)
    a_off = age_x.astype(jnp.int32) * wpr
    p_off = (av + pos_x.astype(jnp.int32)) * wpr
    idx_all = jnp.stack(
        [word_x.reshape(-1, W).astype(jnp.int32),
         a_off.reshape(-1, W),
         p_off.reshape(-1, W)],
        axis=1)
    out = _sc_lookup_sum(word_table, ltab, idx_all, n, ltab.shape[0])
    return out.reshape(b, s, H)
